# baseline (speedup 1.0000x reference)
import jax
import jax.numpy as jnp
from jax import lax
from jax.experimental import pallas as pl
from jax.experimental.pallas import tpu as pltpu
from jax.experimental.pallas import tpu_sc as plsc

N = 10000
DIN = 512
F = 20
FP = 32
K = 16

KNN_BLOCK = 200

SC_NW = 32
SC_QPC = 8
SC_EPC = SC_QPC * K
SC_NCHUNK = N // SC_QPC
SC_TMAX = (SC_NCHUNK + SC_NW - 1) // SC_NW


def _mlp_body(x_ref, w_ref, b_ref, gamma_ref, beta_ref, noise_ref,
              emb_ref, embn_ref):
    h = jnp.dot(x_ref[...], w_ref[...], preferred_element_type=jnp.float32)
    h = h + b_ref[...]
    mean = jnp.mean(h, axis=0, keepdims=True)
    var = jnp.mean((h - mean) ** 2, axis=0, keepdims=True)
    h = (h - mean) / jnp.sqrt(var + 1e-5) * gamma_ref[...] + beta_ref[...]
    e = jnp.maximum(h, 0.0)
    n = x_ref.shape[0]
    z = jnp.zeros((n, FP - F), jnp.float32)
    emb_ref[...] = jnp.concatenate([e, z], axis=1)
    embn_ref[...] = jnp.concatenate([e + noise_ref[...], z], axis=1)


def _mlp(x, w, b, gamma, beta, noise):
    n = x.shape[0]
    out_shape = (jax.ShapeDtypeStruct((n, FP), jnp.float32),
                 jax.ShapeDtypeStruct((n, FP), jnp.float32))
    return pl.pallas_call(_mlp_body, out_shape=out_shape)(
        x, w, b.reshape(1, F), gamma.reshape(1, F), beta.reshape(1, F), noise)


def _knn_body(feat_ref, q_ref, out_ref, d_ref):
    n = feat_ref.shape[0]
    nrows = q_ref.shape[0]
    qc = q_ref[...]
    feat = feat_ref[...]
    sqf = jnp.sum(feat * feat, axis=1)
    sqq = jnp.sum(qc * qc, axis=1)
    g = lax.dot_general(qc, feat, (((1,), (1,)), ((), ())),
                        preferred_element_type=jnp.float32)
    d_ref[...] = sqq[:, None] - 2.0 * g + sqf[None, :]
    col = lax.broadcasted_iota(jnp.int32, (nrows, n), 1)
    rows = (pl.program_id(0) * nrows
            + lax.broadcasted_iota(jnp.int32, (nrows, 1), 0))
    inf = jnp.float32(jnp.inf)
    kiota = lax.broadcasted_iota(jnp.int32, (nrows, K), 1)
    out0 = jnp.zeros((nrows, K), jnp.int32)

    def round_(t, carry):
        out, jprev = carry
        dm = jnp.where(col == jprev, inf, d_ref[...])
        d_ref[...] = dm
        j = jnp.argmin(dm, axis=1).astype(jnp.int32)[:, None]
        out = jnp.where(kiota == t, j, out)
        return out, j

    out, _ = lax.fori_loop(0, K, round_, (out0, rows))
    out_ref[...] = out


def _knn(feat):
    n, dp = feat.shape
    grid = n // KNN_BLOCK
    return pl.pallas_call(
        _knn_body,
        grid=(grid,),
        in_specs=[
            pl.BlockSpec((n, dp), lambda i: (0, 0)),
            pl.BlockSpec((KNN_BLOCK, dp), lambda i: (i, 0)),
        ],
        out_specs=pl.BlockSpec((KNN_BLOCK, K), lambda i: (i, 0)),
        out_shape=jax.ShapeDtypeStruct((n, K), jnp.int32),
        scratch_shapes=[pltpu.VMEM((KNN_BLOCK, n), jnp.float32)],
    )(feat, feat)


def _score_body(embn_hbm, idx_hbm, p_hbm,
                idx_v0, row_v0, tgt_v0, p_v0, sem0,
                idx_v1, row_v1, tgt_v1, p_v1, sem1):
    wid = lax.axis_index("s") * 2 + lax.axis_index("c")
    lane = lax.iota(jnp.int32, 16)
    bufs = ((idx_v0, row_v0, tgt_v0, p_v0, sem0),
            (idx_v1, row_v1, tgt_v1, p_v1, sem1))

    def build_fire(c, buf):
        idx_v, row_v, tgt_v, p_v, sem = buf

        @pl.when(c < SC_NCHUNK)
        def _():
            ebase = c * SC_EPC
            qbase = c * SC_QPC
            pltpu.sync_copy(idx_hbm.at[pl.ds(ebase, SC_EPC)], idx_v)
            pltpu.async_copy(embn_hbm.at[idx_v], row_v, sem)
            pltpu.sync_copy(embn_hbm.at[pl.ds(qbase, SC_QPC)], tgt_v)

    def drain_compute(c, buf):
        idx_v, row_v, tgt_v, p_v, sem = buf

        @pl.when(c < SC_NCHUNK)
        def _():
            ebase = c * SC_EPC
            pltpu.make_async_copy(embn_hbm.at[idx_v], row_v, sem).wait()
            for g in range(SC_QPC):
                b0 = tgt_v[g, pl.ds(0, 16)]
                b1 = tgt_v[g, pl.ds(16, 16)]
                acc = jnp.zeros((16,), jnp.float32)
                for e16 in range(16):
                    e = g * 16 + e16
                    f0 = row_v[e, pl.ds(0, 16)] - b0
                    f1 = row_v[e, pl.ds(16, 16)] - b1
                    pv = f0 * f0 + f1 * f1
                    pv = pv + jnp.take(pv, lane ^ 8)
                    pv = pv + jnp.take(pv, lane ^ 4)
                    pv = pv + jnp.take(pv, lane ^ 2)
                    pv = pv + jnp.take(pv, lane ^ 1)
                    acc = jnp.where(lane == e16, pv, acc)
                p_v[pl.ds(g * 16, 16)] = acc
            pltpu.sync_copy(p_v, p_hbm.at[pl.ds(ebase, SC_EPC)])

    build_fire(wid, bufs[0])

    def pair(u, carry):
        t0 = 2 * u
        build_fire((t0 + 1) * SC_NW + wid, bufs[1])
        drain_compute(t0 * SC_NW + wid, bufs[0])
        build_fire((t0 + 2) * SC_NW + wid, bufs[0])
        drain_compute((t0 + 1) * SC_NW + wid, bufs[1])
        return carry

    lax.fori_loop(0, SC_TMAX // 2, pair, 0)


def _score(embn, idx):
    mesh = plsc.VectorSubcoreMesh(core_axis_name="c", subcore_axis_name="s")
    buf_scratch = [
        pltpu.VMEM((SC_EPC,), jnp.int32),
        pltpu.VMEM((SC_EPC, FP), jnp.float32),
        pltpu.VMEM((SC_QPC, FP), jnp.float32),
        pltpu.VMEM((SC_EPC,), jnp.float32),
        pltpu.SemaphoreType.DMA,
    ]
    kern = pl.kernel(
        _score_body,
        out_type=jax.ShapeDtypeStruct((N * K,), jnp.float32),
        mesh=mesh,
        compiler_params=pltpu.CompilerParams(use_tc_tiling_on_sc=False),
        scratch_types=buf_scratch + buf_scratch,
    )
    return kern(embn, idx)


def _finish_body(s_ref, p_ref):
    p_ref[...] = jnp.exp(-jnp.sqrt(s_ref[...]))


def _finish(s2d):
    return pl.pallas_call(
        _finish_body,
        out_shape=jax.ShapeDtypeStruct(s2d.shape, jnp.float32),
    )(s2d)


def kernel(x, pos, batch, W, b, gamma, beta):
    noise = jax.random.uniform(jax.random.key(42), (N, F), jnp.float32) * 1e-4
    emb, embn = _mlp(x, W, b, gamma, beta, noise)
    idx_emb = _knn(emb)
    idx_pos = _knn(pos)
    src = idx_emb.reshape(N * K)
    tgt = jnp.repeat(jnp.arange(N, dtype=jnp.int32), K)
    s = _score(embn, src)
    p = _finish(s.reshape(N * K // 128, 128)).reshape(N * K)
    edges_large = jnp.stack([src, tgt], axis=0)
    soft_index_v = jnp.stack([p, tgt.astype(jnp.float32)], axis=0)
    pos_edges = jnp.stack([idx_pos.reshape(N * K), tgt], axis=0)
    edge_index = jnp.concatenate([edges_large, pos_edges], axis=1)
    return edges_large, soft_index_v, edge_index


# speedup vs baseline: 1.0045x; 1.0045x over previous
import jax
import jax.numpy as jnp
from jax import lax
from jax.experimental import pallas as pl
from jax.experimental.pallas import tpu as pltpu
from jax.experimental.pallas import tpu_sc as plsc

N = 10000
DIN = 512
F = 20
FP = 32
K = 16

KNN_BLOCK = 200

SC_NW = 32
SC_QPC = 8
SC_EPC = SC_QPC * K
SC_NCHUNK = N // SC_QPC
SC_TMAX = (SC_NCHUNK + SC_NW - 1) // SC_NW


def _mlp_body(x_ref, w_ref, b_ref, gamma_ref, beta_ref, noise_ref,
              emb_ref, embn_ref):
    h = jnp.dot(x_ref[...], w_ref[...], preferred_element_type=jnp.float32)
    h = h + b_ref[...]
    mean = jnp.mean(h, axis=0, keepdims=True)
    var = jnp.mean((h - mean) ** 2, axis=0, keepdims=True)
    h = (h - mean) / jnp.sqrt(var + 1e-5) * gamma_ref[...] + beta_ref[...]
    e = jnp.maximum(h, 0.0)
    n = x_ref.shape[0]
    z = jnp.zeros((n, FP - F), jnp.float32)
    emb_ref[...] = jnp.concatenate([e, z], axis=1)
    embn_ref[...] = jnp.concatenate([e + noise_ref[...], z], axis=1)


def _mlp(x, w, b, gamma, beta, noise):
    n = x.shape[0]
    out_shape = (jax.ShapeDtypeStruct((n, FP), jnp.float32),
                 jax.ShapeDtypeStruct((n, FP), jnp.float32))
    return pl.pallas_call(_mlp_body, out_shape=out_shape)(
        x, w, b.reshape(1, F), gamma.reshape(1, F), beta.reshape(1, F), noise)


def _knn_body(feat_ref, q_ref, out_ref, d_ref):
    n = feat_ref.shape[0]
    nrows = q_ref.shape[0]
    qc = q_ref[...]
    feat = feat_ref[...]
    sqf = jnp.sum(feat * feat, axis=1)
    sqq = jnp.sum(qc * qc, axis=1)
    g = lax.dot_general(qc, feat, (((1,), (1,)), ((), ())),
                        preferred_element_type=jnp.float32)
    d_ref[...] = sqq[:, None] - 2.0 * g + sqf[None, :]
    col = lax.broadcasted_iota(jnp.int32, (nrows, n), 1)
    rows = (pl.program_id(0) * nrows
            + lax.broadcasted_iota(jnp.int32, (nrows, 1), 0))
    inf = jnp.float32(jnp.inf)
    kiota = lax.broadcasted_iota(jnp.int32, (nrows, K), 1)
    out0 = jnp.zeros((nrows, K), jnp.int32)

    def round_(t, carry):
        out, jprev = carry
        dm = jnp.where(col == jprev, inf, d_ref[...])
        d_ref[...] = dm
        m = jnp.min(dm, axis=1, keepdims=True)
        j = jnp.min(jnp.where(dm == m, col, n), axis=1, keepdims=True)
        out = jnp.where(kiota == t, j, out)
        return out, j

    out, _ = lax.fori_loop(0, K, round_, (out0, rows))
    out_ref[...] = out


def _knn(feat):
    n, dp = feat.shape
    grid = n // KNN_BLOCK
    return pl.pallas_call(
        _knn_body,
        grid=(grid,),
        in_specs=[
            pl.BlockSpec((n, dp), lambda i: (0, 0)),
            pl.BlockSpec((KNN_BLOCK, dp), lambda i: (i, 0)),
        ],
        out_specs=pl.BlockSpec((KNN_BLOCK, K), lambda i: (i, 0)),
        out_shape=jax.ShapeDtypeStruct((n, K), jnp.int32),
        scratch_shapes=[pltpu.VMEM((KNN_BLOCK, n), jnp.float32)],
    )(feat, feat)


def _score_body(embn_hbm, idx_hbm, p_hbm,
                idx_v0, row_v0, tgt_v0, p_v0, sem0,
                idx_v1, row_v1, tgt_v1, p_v1, sem1):
    wid = lax.axis_index("s") * 2 + lax.axis_index("c")
    lane = lax.iota(jnp.int32, 16)
    bufs = ((idx_v0, row_v0, tgt_v0, p_v0, sem0),
            (idx_v1, row_v1, tgt_v1, p_v1, sem1))

    def build_fire(c, buf):
        idx_v, row_v, tgt_v, p_v, sem = buf

        @pl.when(c < SC_NCHUNK)
        def _():
            ebase = c * SC_EPC
            qbase = c * SC_QPC
            pltpu.sync_copy(idx_hbm.at[pl.ds(ebase, SC_EPC)], idx_v)
            pltpu.async_copy(embn_hbm.at[idx_v], row_v, sem)
            pltpu.sync_copy(embn_hbm.at[pl.ds(qbase, SC_QPC)], tgt_v)

    def drain_compute(c, buf):
        idx_v, row_v, tgt_v, p_v, sem = buf

        @pl.when(c < SC_NCHUNK)
        def _():
            ebase = c * SC_EPC
            pltpu.make_async_copy(embn_hbm.at[idx_v], row_v, sem).wait()
            for g in range(SC_QPC):
                b0 = tgt_v[g, pl.ds(0, 16)]
                b1 = tgt_v[g, pl.ds(16, 16)]
                acc = jnp.zeros((16,), jnp.float32)
                for e16 in range(16):
                    e = g * 16 + e16
                    f0 = row_v[e, pl.ds(0, 16)] - b0
                    f1 = row_v[e, pl.ds(16, 16)] - b1
                    pv = f0 * f0 + f1 * f1
                    pv = pv + jnp.take(pv, lane ^ 8)
                    pv = pv + jnp.take(pv, lane ^ 4)
                    pv = pv + jnp.take(pv, lane ^ 2)
                    pv = pv + jnp.take(pv, lane ^ 1)
                    acc = jnp.where(lane == e16, pv, acc)
                p_v[pl.ds(g * 16, 16)] = acc
            pltpu.sync_copy(p_v, p_hbm.at[pl.ds(ebase, SC_EPC)])

    build_fire(wid, bufs[0])

    def pair(u, carry):
        t0 = 2 * u
        build_fire((t0 + 1) * SC_NW + wid, bufs[1])
        drain_compute(t0 * SC_NW + wid, bufs[0])
        build_fire((t0 + 2) * SC_NW + wid, bufs[0])
        drain_compute((t0 + 1) * SC_NW + wid, bufs[1])
        return carry

    lax.fori_loop(0, SC_TMAX // 2, pair, 0)


def _score(embn, idx):
    mesh = plsc.VectorSubcoreMesh(core_axis_name="c", subcore_axis_name="s")
    buf_scratch = [
        pltpu.VMEM((SC_EPC,), jnp.int32),
        pltpu.VMEM((SC_EPC, FP), jnp.float32),
        pltpu.VMEM((SC_QPC, FP), jnp.float32),
        pltpu.VMEM((SC_EPC,), jnp.float32),
        pltpu.SemaphoreType.DMA,
    ]
    kern = pl.kernel(
        _score_body,
        out_type=jax.ShapeDtypeStruct((N * K,), jnp.float32),
        mesh=mesh,
        compiler_params=pltpu.CompilerParams(use_tc_tiling_on_sc=False),
        scratch_types=buf_scratch + buf_scratch,
    )
    return kern(embn, idx)


def _finish_body(s_ref, p_ref):
    p_ref[...] = jnp.exp(-jnp.sqrt(s_ref[...]))


def _finish(s2d):
    return pl.pallas_call(
        _finish_body,
        out_shape=jax.ShapeDtypeStruct(s2d.shape, jnp.float32),
    )(s2d)


def kernel(x, pos, batch, W, b, gamma, beta):
    noise = jax.random.uniform(jax.random.key(42), (N, F), jnp.float32) * 1e-4
    emb, embn = _mlp(x, W, b, gamma, beta, noise)
    idx_emb = _knn(emb)
    idx_pos = _knn(pos)
    src = idx_emb.reshape(N * K)
    tgt = jnp.repeat(jnp.arange(N, dtype=jnp.int32), K)
    s = _score(embn, src)
    p = _finish(s.reshape(N * K // 128, 128)).reshape(N * K)
    edges_large = jnp.stack([src, tgt], axis=0)
    soft_index_v = jnp.stack([p, tgt.astype(jnp.float32)], axis=0)
    pos_edges = jnp.stack([idx_pos.reshape(N * K), tgt], axis=0)
    edge_index = jnp.concatenate([edges_large, pos_edges], axis=1)
    return edges_large, soft_index_v, edge_index


# speedup vs baseline: 1.0640x; 1.0593x over previous
import jax
import jax.numpy as jnp
from jax import lax
from jax.experimental import pallas as pl
from jax.experimental.pallas import tpu as pltpu
from jax.experimental.pallas import tpu_sc as plsc

N = 10000
DIN = 512
F = 20
FP = 32
K = 16

KNN_BLOCK = 400

SC_NW = 32
SC_QPC = 8
SC_EPC = SC_QPC * K
SC_NCHUNK = N // SC_QPC
SC_TMAX = (SC_NCHUNK + SC_NW - 1) // SC_NW


def _mlp_body(x_ref, w_ref, b_ref, gamma_ref, beta_ref, noise_ref,
              emb_ref, embn_ref):
    h = jnp.dot(x_ref[...], w_ref[...], preferred_element_type=jnp.float32)
    h = h + b_ref[...]
    mean = jnp.mean(h, axis=0, keepdims=True)
    var = jnp.mean((h - mean) ** 2, axis=0, keepdims=True)
    h = (h - mean) / jnp.sqrt(var + 1e-5) * gamma_ref[...] + beta_ref[...]
    e = jnp.maximum(h, 0.0)
    n = x_ref.shape[0]
    z = jnp.zeros((n, FP - F), jnp.float32)
    emb_ref[...] = jnp.concatenate([e, z], axis=1)
    embn_ref[...] = jnp.concatenate([e + noise_ref[...], z], axis=1)


def _mlp(x, w, b, gamma, beta, noise):
    n = x.shape[0]
    out_shape = (jax.ShapeDtypeStruct((n, FP), jnp.float32),
                 jax.ShapeDtypeStruct((n, FP), jnp.float32))
    return pl.pallas_call(_mlp_body, out_shape=out_shape)(
        x, w, b.reshape(1, F), gamma.reshape(1, F), beta.reshape(1, F), noise)


def _knn_body(feat_ref, q_ref, out_ref, d_ref):
    n = feat_ref.shape[0]
    nrows = q_ref.shape[0]
    qc = q_ref[...]
    feat = feat_ref[...]
    sqf = jnp.sum(feat * feat, axis=1)
    sqq = jnp.sum(qc * qc, axis=1)
    g = lax.dot_general(qc, feat, (((1,), (1,)), ((), ())),
                        preferred_element_type=jnp.float32)
    d_ref[...] = sqq[:, None] - 2.0 * g + sqf[None, :]
    col = lax.broadcasted_iota(jnp.int32, (nrows, n), 1)
    rows = (pl.program_id(0) * nrows
            + lax.broadcasted_iota(jnp.int32, (nrows, 1), 0))
    inf = jnp.float32(jnp.inf)
    kiota = lax.broadcasted_iota(jnp.int32, (nrows, K), 1)
    out0 = jnp.zeros((nrows, K), jnp.int32)

    def round_(t, carry):
        out, jprev = carry
        dm = jnp.where(col == jprev, inf, d_ref[...])
        d_ref[...] = dm
        m = jnp.min(dm, axis=1, keepdims=True)
        j = jnp.min(jnp.where(dm == m, col, n), axis=1, keepdims=True)
        out = jnp.where(kiota == t, j, out)
        return out, j

    out, _ = lax.fori_loop(0, K, round_, (out0, rows))
    out_ref[...] = out


def _knn(feat):
    n, dp = feat.shape
    grid = n // KNN_BLOCK
    return pl.pallas_call(
        _knn_body,
        grid=(grid,),
        in_specs=[
            pl.BlockSpec((n, dp), lambda i: (0, 0)),
            pl.BlockSpec((KNN_BLOCK, dp), lambda i: (i, 0)),
        ],
        out_specs=pl.BlockSpec((KNN_BLOCK, K), lambda i: (i, 0)),
        out_shape=jax.ShapeDtypeStruct((n, K), jnp.int32),
        scratch_shapes=[pltpu.VMEM((KNN_BLOCK, n), jnp.float32)],
    )(feat, feat)


def _score_body(embn_hbm, idx_hbm, p_hbm,
                idx_v0, row_v0, tgt_v0, p_v0, sem0,
                idx_v1, row_v1, tgt_v1, p_v1, sem1):
    wid = lax.axis_index("s") * 2 + lax.axis_index("c")
    lane = lax.iota(jnp.int32, 16)
    bufs = ((idx_v0, row_v0, tgt_v0, p_v0, sem0),
            (idx_v1, row_v1, tgt_v1, p_v1, sem1))

    def build_fire(c, buf):
        idx_v, row_v, tgt_v, p_v, sem = buf

        @pl.when(c < SC_NCHUNK)
        def _():
            ebase = c * SC_EPC
            qbase = c * SC_QPC
            pltpu.sync_copy(idx_hbm.at[pl.ds(ebase, SC_EPC)], idx_v)
            pltpu.async_copy(embn_hbm.at[idx_v], row_v, sem)
            pltpu.sync_copy(embn_hbm.at[pl.ds(qbase, SC_QPC)], tgt_v)

    def drain_compute(c, buf):
        idx_v, row_v, tgt_v, p_v, sem = buf

        @pl.when(c < SC_NCHUNK)
        def _():
            ebase = c * SC_EPC
            pltpu.make_async_copy(embn_hbm.at[idx_v], row_v, sem).wait()
            for g in range(SC_QPC):
                b0 = tgt_v[g, pl.ds(0, 16)]
                b1 = tgt_v[g, pl.ds(16, 16)]
                acc = jnp.zeros((16,), jnp.float32)
                for e16 in range(16):
                    e = g * 16 + e16
                    f0 = row_v[e, pl.ds(0, 16)] - b0
                    f1 = row_v[e, pl.ds(16, 16)] - b1
                    pv = f0 * f0 + f1 * f1
                    pv = pv + jnp.take(pv, lane ^ 8)
                    pv = pv + jnp.take(pv, lane ^ 4)
                    pv = pv + jnp.take(pv, lane ^ 2)
                    pv = pv + jnp.take(pv, lane ^ 1)
                    acc = jnp.where(lane == e16, pv, acc)
                p_v[pl.ds(g * 16, 16)] = acc
            pltpu.sync_copy(p_v, p_hbm.at[pl.ds(ebase, SC_EPC)])

    build_fire(wid, bufs[0])

    def pair(u, carry):
        t0 = 2 * u
        build_fire((t0 + 1) * SC_NW + wid, bufs[1])
        drain_compute(t0 * SC_NW + wid, bufs[0])
        build_fire((t0 + 2) * SC_NW + wid, bufs[0])
        drain_compute((t0 + 1) * SC_NW + wid, bufs[1])
        return carry

    lax.fori_loop(0, SC_TMAX // 2, pair, 0)


def _score(embn, idx):
    mesh = plsc.VectorSubcoreMesh(core_axis_name="c", subcore_axis_name="s")
    buf_scratch = [
        pltpu.VMEM((SC_EPC,), jnp.int32),
        pltpu.VMEM((SC_EPC, FP), jnp.float32),
        pltpu.VMEM((SC_QPC, FP), jnp.float32),
        pltpu.VMEM((SC_EPC,), jnp.float32),
        pltpu.SemaphoreType.DMA,
    ]
    kern = pl.kernel(
        _score_body,
        out_type=jax.ShapeDtypeStruct((N * K,), jnp.float32),
        mesh=mesh,
        compiler_params=pltpu.CompilerParams(use_tc_tiling_on_sc=False),
        scratch_types=buf_scratch + buf_scratch,
    )
    return kern(embn, idx)


def _finish_body(s_ref, p_ref):
    p_ref[...] = jnp.exp(-jnp.sqrt(s_ref[...]))


def _finish(s2d):
    return pl.pallas_call(
        _finish_body,
        out_shape=jax.ShapeDtypeStruct(s2d.shape, jnp.float32),
    )(s2d)


def kernel(x, pos, batch, W, b, gamma, beta):
    noise = jax.random.uniform(jax.random.key(42), (N, F), jnp.float32) * 1e-4
    emb, embn = _mlp(x, W, b, gamma, beta, noise)
    idx_emb = _knn(emb)
    idx_pos = _knn(pos)
    src = idx_emb.reshape(N * K)
    tgt = jnp.repeat(jnp.arange(N, dtype=jnp.int32), K)
    s = _score(embn, src)
    p = _finish(s.reshape(N * K // 128, 128)).reshape(N * K)
    edges_large = jnp.stack([src, tgt], axis=0)
    soft_index_v = jnp.stack([p, tgt.astype(jnp.float32)], axis=0)
    pos_edges = jnp.stack([idx_pos.reshape(N * K), tgt], axis=0)
    edge_index = jnp.concatenate([edges_large, pos_edges], axis=1)
    return edges_large, soft_index_v, edge_index
